# baseline (speedup 1.0000x reference)
import functools

import jax
import jax.numpy as jnp
from jax import lax
from jax.experimental import pallas as pl
from jax.experimental.pallas import tpu as pltpu
from jax.experimental.pallas import tpu_sc as plsc

NC = 2
NS = 16
NW = NC * NS
CK = 128
SB = 16
NB = 3
RB = 2000
NG = 8


def _acc_rows(n):
    per_tile = -(-(n + 1) // (NS * 8)) * 8
    return NS * per_tile


def _fill(buf, val):
    def row(i, carry):
        for j in range(128 // 16):
            buf[i, pl.ds(j * 16, 16)] = jnp.full((16,), val, jnp.float32)
        return carry
    lax.fori_loop(0, CK, row, 0)


def _zero_acc(zbuf, acc_sh, s, out_rows):
    for q in range(0, out_rows, CK):
        step = min(CK, out_rows - q)
        pltpu.sync_copy(zbuf.at[pl.ds(0, step), :],
                        acc_sh.at[pl.ds(s * out_rows + q, step), :])


@functools.lru_cache(maxsize=None)
def _make_degree(n, e_pad):
    rows = _acc_rows(n)
    chunk_rows_per_w = e_pad // NW // CK
    n_super = chunk_rows_per_w // SB
    out_rows = rows // NS
    mesh = plsc.VectorSubcoreMesh(core_axis_name="c", subcore_axis_name="s")

    def body(dst_hbm, out_hbm, zbuf, ones_v, idx_d,
             ss0, ss1, ss2, ss3, acc_sh):
        c = lax.axis_index("c")
        s = lax.axis_index("s")
        ssems = [ss0, ss1, ss2, ss3]

        _fill(zbuf, 0.0)
        _fill(ones_v, 1.0)
        _zero_acc(zbuf, acc_sh, s, out_rows)
        plsc.subcore_barrier()

        row0 = (c * NS + s) * chunk_rows_per_w

        def super_step(t, carry):
            pltpu.sync_copy(dst_hbm.at[pl.ds(row0 + t * SB, SB), :], idx_d)
            sd = [None] * SB
            for b in range(SB):
                if b >= NB:
                    sd[b - NB].wait()
                sd[b] = pltpu.async_copy(ones_v, acc_sh.at[idx_d.at[b]],
                                         ssems[b % NB], add=True)
            for b in range(SB - NB, SB):
                sd[b].wait()
            return carry

        lax.fori_loop(0, n_super, super_step, 0)
        plsc.subcore_barrier()
        pltpu.sync_copy(acc_sh.at[pl.ds(s * out_rows, out_rows), :],
                        out_hbm.at[c].at[pl.ds(s * out_rows, out_rows), :])

    return pl.kernel(
        body,
        out_type=jax.ShapeDtypeStruct((NC, rows, 128), jnp.float32),
        mesh=mesh,
        scratch_types=[
            pltpu.VMEM((CK, 128), jnp.float32),
            pltpu.VMEM((CK, 128), jnp.float32),
            pltpu.VMEM((SB, CK), jnp.int32),
            pltpu.SemaphoreType.DMA,
            pltpu.SemaphoreType.DMA,
            pltpu.SemaphoreType.DMA,
            pltpu.SemaphoreType.DMA,
            pltpu.VMEM_SHARED((rows, 128), jnp.float32),
        ],
    )


@functools.lru_cache(maxsize=None)
def _make_aggregate(n, feat, e_pad, m0, m1):
    rows = _acc_rows(n)
    out_rows = rows // NS
    mesh = plsc.VectorSubcoreMesh(core_axis_name="c", subcore_axis_name="s")

    def body(g_hbm, src_hbm, dst_hbm, out_hbm,
             ig0, ig1, ig2, id0, id1, id2, rv0, rv1, rv2,
             es0, es1, es2, ed0, ed1, ed2,
             gs0, gs1, gs2, ss0, ss1, ss2, acc_sh):
        c = lax.axis_index("c")
        s = lax.axis_index("s")
        idx_g = [ig0, ig1, ig2]
        idx_d = [id0, id1, id2]
        rows_v = [rv0, rv1, rv2]
        isems = [es0, es1, es2]
        dsems = [ed0, ed1, ed2]
        gsems = [gs0, gs1, gs2]
        ssems = [ss0, ss1, ss2]

        _fill(rv0, 0.0)
        _zero_acc(rv0, acc_sh, s, out_rows)
        plsc.subcore_barrier()

        row0 = jnp.where(c == 0, s * (m0 * SB),
                         NS * m0 * SB + s * (m1 * SB))
        n_super = jnp.where(c == 0, m0, m1)

        def super_step(t, carry):
            e0 = (row0 + t * SB) * CK
            ids = [None] * SB
            idd = [None] * SB
            gd = [None] * SB
            sd = [None] * SB

            def load_idx(b):
                r = b % NB
                ids[b] = pltpu.async_copy(
                    src_hbm.at[pl.ds(e0 + b * CK, CK)], idx_g[r], isems[r])
                idd[b] = pltpu.async_copy(
                    dst_hbm.at[pl.ds(e0 + b * CK, CK)], idx_d[r], dsems[r])

            def gather(b):
                r = b % NB
                ids[b].wait()
                gd[b] = pltpu.async_copy(g_hbm.at[idx_g[r]], rows_v[r],
                                         gsems[r])

            def scatter(b):
                r = b % NB
                gd[b].wait()
                idd[b].wait()
                sd[b] = pltpu.async_copy(rows_v[r], acc_sh.at[idx_d[r]],
                                         ssems[r], add=True)

            for b in range(SB):
                if b >= NB:
                    sd[b - NB].wait()
                load_idx(b)
                if b >= 1:
                    gather(b - 1)
                if b >= 2:
                    scatter(b - 2)
            gather(SB - 1)
            scatter(SB - 2)
            scatter(SB - 1)
            for b in range(SB - NB, SB):
                sd[b].wait()
            return carry

        lax.fori_loop(0, n_super, super_step, 0)
        plsc.subcore_barrier()
        pltpu.sync_copy(acc_sh.at[pl.ds(s * out_rows, out_rows), :],
                        out_hbm.at[c].at[pl.ds(s * out_rows, out_rows), :])

    return pl.kernel(
        body,
        out_type=jax.ShapeDtypeStruct((NC, rows, feat), jnp.float32),
        mesh=mesh,
        scratch_types=(
            [pltpu.VMEM((CK,), jnp.int32) for _ in range(2 * NB)]
            + [pltpu.VMEM((CK, feat), jnp.float32) for _ in range(NB)]
            + [pltpu.SemaphoreType.DMA for _ in range(4 * NB)]
            + [pltpu.VMEM_SHARED((rows, feat), jnp.float32)]
        ),
    )


def _first_layer_body(x_ref, w_ref, deg_ref, g_ref, dinv_ref):
    deg = deg_ref[0, :, 0:1] + deg_ref[1, :, 0:1] + 1.0
    dinv = lax.rsqrt(deg)
    g_ref[...] = jnp.dot(x_ref[...], w_ref[...],
                         preferred_element_type=jnp.float32) * dinv
    dinv_ref[...] = jnp.broadcast_to(dinv, dinv_ref.shape)


def _first_layer(x, w, deg_parts):
    n, d = x.shape
    return pl.pallas_call(
        _first_layer_body,
        grid=(n // RB,),
        in_specs=[
            pl.BlockSpec((RB, d), lambda i: (i, 0)),
            pl.BlockSpec((d, w.shape[1]), lambda i: (0, 0)),
            pl.BlockSpec((NC, RB, 128), lambda i: (0, i, 0)),
        ],
        out_specs=[
            pl.BlockSpec((RB, w.shape[1]), lambda i: (i, 0)),
            pl.BlockSpec((RB, 16), lambda i: (i, 0)),
        ],
        out_shape=[
            jax.ShapeDtypeStruct((n, w.shape[1]), jnp.float32),
            jax.ShapeDtypeStruct((n, 16), jnp.float32),
        ],
    )(x, w, deg_parts)


def _mid_layer_body(s_ref, gp_ref, dinv_ref, b_ref, w_ref, g_ref):
    dinv = dinv_ref[:, 0:1]
    act = jnp.maximum(
        dinv * (s_ref[0] + s_ref[1] + gp_ref[...]) + b_ref[...], 0.0)
    g_ref[...] = jnp.dot(act, w_ref[...],
                         preferred_element_type=jnp.float32) * dinv


def _mid_layer(s_parts, g_prev, dinv16, b, w):
    n, d = g_prev.shape
    return pl.pallas_call(
        _mid_layer_body,
        grid=(n // RB,),
        in_specs=[
            pl.BlockSpec((NC, RB, d), lambda i: (0, i, 0)),
            pl.BlockSpec((RB, d), lambda i: (i, 0)),
            pl.BlockSpec((RB, 16), lambda i: (i, 0)),
            pl.BlockSpec((1, d), lambda i: (0, 0)),
            pl.BlockSpec((d, w.shape[1]), lambda i: (0, 0)),
        ],
        out_specs=pl.BlockSpec((RB, w.shape[1]), lambda i: (i, 0)),
        out_shape=jax.ShapeDtypeStruct((n, w.shape[1]), jnp.float32),
    )(s_parts, g_prev, dinv16, b.reshape(1, -1), w)


def _pool_body(s_ref, gp_ref, dinv_ref, b_ref, batch_ref, fcw_ref, fcb_ref,
               out_ref, sums, cnts):
    i = pl.program_id(0)

    @pl.when(i == 0)
    def _():
        sums[...] = jnp.zeros_like(sums)
        cnts[...] = jnp.zeros_like(cnts)

    dinv = dinv_ref[:, 0:1]
    h = jnp.maximum(
        dinv * (s_ref[0] + s_ref[1] + gp_ref[...]) + b_ref[...], 0.0)
    onehot = (batch_ref[...] ==
              lax.broadcasted_iota(jnp.int32, (1, NG), 1)).astype(jnp.float32)
    dims = (((0,), (0,)), ((), ()))
    sums[...] += lax.dot_general(onehot, h, dims,
                                 preferred_element_type=jnp.float32)
    cnts[...] += lax.dot_general(onehot, jnp.ones_like(h), dims,
                                 preferred_element_type=jnp.float32)

    @pl.when(i == pl.num_programs(0) - 1)
    def _():
        pooled = sums[...] / jnp.maximum(cnts[...], 1.0)
        z = jnp.dot(pooled, fcw_ref[...],
                    preferred_element_type=jnp.float32) + fcb_ref[...]
        out_ref[...] = jax.nn.sigmoid(z)


def _pool(s_parts, g_prev, dinv16, b, batch, fcw, fcb):
    n, d = g_prev.shape
    return pl.pallas_call(
        _pool_body,
        grid=(n // RB,),
        in_specs=[
            pl.BlockSpec((NC, RB, d), lambda i: (0, i, 0)),
            pl.BlockSpec((RB, d), lambda i: (i, 0)),
            pl.BlockSpec((RB, 16), lambda i: (i, 0)),
            pl.BlockSpec((1, d), lambda i: (0, 0)),
            pl.BlockSpec((RB, 1), lambda i: (i, 0)),
            pl.BlockSpec((d, 1), lambda i: (0, 0)),
            pl.BlockSpec((1, 1), lambda i: (0, 0)),
        ],
        out_specs=pl.BlockSpec((NG, 1), lambda i: (0, 0)),
        out_shape=jax.ShapeDtypeStruct((NG, 1), jnp.float32),
        scratch_shapes=[
            pltpu.VMEM((NG, d), jnp.float32),
            pltpu.VMEM((NG, d), jnp.float32),
        ],
    )(s_parts, g_prev, dinv16, b.reshape(1, -1), batch.reshape(-1, 1),
      fcw, fcb.reshape(1, 1))


def kernel(x, edge_index, batch, W1, b1, W2, b2, W3, b3, fcW, fcb):
    n, d = x.shape
    e = edge_index.shape[1]
    e_pad = -(-e // (NW * CK * SB)) * (NW * CK * SB)
    pad = e_pad - e
    src_s, dst_s = lax.sort_key_val(edge_index[0], edge_index[1])
    src = jnp.concatenate([src_s, jnp.zeros((pad,), jnp.int32)])
    dst = jnp.concatenate([dst_s, jnp.full((pad,), n, jnp.int32)])

    deg_parts = _make_degree(n, e_pad)(dst.reshape(e_pad // CK, CK))
    m_total = e_pad // (NS * SB * CK)
    agg = _make_aggregate(n, d, e_pad, 9, m_total - 9)

    g1, dinv16 = _first_layer(x, W1, deg_parts)
    s1 = agg(g1, src, dst)
    g2 = _mid_layer(s1, g1, dinv16, b1, W2)
    s2 = agg(g2, src, dst)
    g3 = _mid_layer(s2, g2, dinv16, b2, W3)
    s3 = agg(g3, src, dst)
    return _pool(s3, g3, dinv16, b3, batch, fcW, fcb)


# speedup vs baseline: 1.3396x; 1.3396x over previous
import functools

import jax
import jax.numpy as jnp
from jax import lax
from jax.experimental import pallas as pl
from jax.experimental.pallas import tpu as pltpu
from jax.experimental.pallas import tpu_sc as plsc

NC = 2
NS = 16
NW = NC * NS
CK = 128
SB = 16
NB = 3
RB = 2000
NG = 8


def _acc_rows(n):
    per_tile = -(-(n + 1) // (NS * 8)) * 8
    return NS * per_tile


def _fill(buf, val):
    def row(i, carry):
        for j in range(128 // 16):
            buf[i, pl.ds(j * 16, 16)] = jnp.full((16,), val, jnp.float32)
        return carry
    lax.fori_loop(0, CK, row, 0)


def _zero_acc(zbuf, acc_sh, s, out_rows):
    for q in range(0, out_rows, CK):
        step = min(CK, out_rows - q)
        pltpu.sync_copy(zbuf.at[pl.ds(0, step), :],
                        acc_sh.at[pl.ds(s * out_rows + q, step), :])


@functools.lru_cache(maxsize=None)
def _make_degree(n, e_pad):
    rows = _acc_rows(n)
    chunk_rows_per_w = e_pad // NW // CK
    n_super = chunk_rows_per_w // SB
    out_rows = rows // NS
    mesh = plsc.VectorSubcoreMesh(core_axis_name="c", subcore_axis_name="s")

    def body(dst_hbm, out_hbm, zbuf, ones_v, idx_d,
             ss0, ss1, ss2, ss3, acc_sh):
        c = lax.axis_index("c")
        s = lax.axis_index("s")
        ssems = [ss0, ss1, ss2, ss3]

        _fill(zbuf, 0.0)
        _fill(ones_v, 1.0)
        _zero_acc(zbuf, acc_sh, s, out_rows)
        plsc.subcore_barrier()

        row0 = (c * NS + s) * chunk_rows_per_w

        def super_step(t, carry):
            pltpu.sync_copy(dst_hbm.at[pl.ds(row0 + t * SB, SB), :], idx_d)
            sd = [None] * SB
            for b in range(SB):
                if b >= NB:
                    sd[b - NB].wait()
                sd[b] = pltpu.async_copy(ones_v, acc_sh.at[idx_d.at[b]],
                                         ssems[b % NB], add=True)
            for b in range(SB - NB, SB):
                sd[b].wait()
            return carry

        lax.fori_loop(0, n_super, super_step, 0)
        plsc.subcore_barrier()
        pltpu.sync_copy(acc_sh.at[pl.ds(s * out_rows, out_rows), :],
                        out_hbm.at[c].at[pl.ds(s * out_rows, out_rows), :])

    return pl.kernel(
        body,
        out_type=jax.ShapeDtypeStruct((NC, rows, 128), jnp.float32),
        mesh=mesh,
        scratch_types=[
            pltpu.VMEM((CK, 128), jnp.float32),
            pltpu.VMEM((CK, 128), jnp.float32),
            pltpu.VMEM((SB, CK), jnp.int32),
            pltpu.SemaphoreType.DMA,
            pltpu.SemaphoreType.DMA,
            pltpu.SemaphoreType.DMA,
            pltpu.SemaphoreType.DMA,
            pltpu.VMEM_SHARED((rows, 128), jnp.float32),
        ],
    )


@functools.lru_cache(maxsize=None)
def _make_aggregate(n, feat, e_pad, m0, m1):
    rows = _acc_rows(n)
    out_rows = rows // NS
    mesh = plsc.VectorSubcoreMesh(core_axis_name="c", subcore_axis_name="s")

    def body(g_hbm, src_hbm, dst_hbm, out_hbm,
             ig0, ig1, ig2, id0, id1, id2, rv0, rv1, rv2,
             es0, es1, es2, ed0, ed1, ed2,
             gs0, gs1, gs2, ss0, ss1, ss2, acc_sh):
        c = lax.axis_index("c")
        s = lax.axis_index("s")
        idx_g = [ig0, ig1, ig2]
        idx_d = [id0, id1, id2]
        rows_v = [rv0, rv1, rv2]
        isems = [es0, es1, es2]
        dsems = [ed0, ed1, ed2]
        gsems = [gs0, gs1, gs2]
        ssems = [ss0, ss1, ss2]

        _fill(rv0, 0.0)
        _zero_acc(rv0, acc_sh, s, out_rows)
        plsc.subcore_barrier()

        row0 = jnp.where(c == 0, s * (m0 * SB),
                         NS * m0 * SB + s * (m1 * SB))
        n_super = jnp.where(c == 0, m0, m1)

        def super_step(t, carry):
            e0 = (row0 + t * SB) * CK
            ids = [None] * SB
            idd = [None] * SB
            gd = [None] * SB
            sd = [None] * SB

            def load_idx(b):
                r = b % NB
                ids[b] = pltpu.async_copy(
                    src_hbm.at[pl.ds(e0 + b * CK, CK)], idx_g[r], isems[r])
                idd[b] = pltpu.async_copy(
                    dst_hbm.at[pl.ds(e0 + b * CK, CK)], idx_d[r], dsems[r])

            def gather(b):
                r = b % NB
                ids[b].wait()
                gd[b] = pltpu.async_copy(g_hbm.at[idx_g[r]], rows_v[r],
                                         gsems[r])

            def scatter(b):
                r = b % NB
                gd[b].wait()
                idd[b].wait()
                sd[b] = pltpu.async_copy(rows_v[r], acc_sh.at[idx_d[r]],
                                         ssems[r], add=True)

            for b in range(SB):
                if b >= NB:
                    sd[b - NB].wait()
                load_idx(b)
                if b >= 1:
                    gather(b - 1)
                if b >= 2:
                    scatter(b - 2)
            gather(SB - 1)
            scatter(SB - 2)
            scatter(SB - 1)
            for b in range(SB - NB, SB):
                sd[b].wait()
            return carry

        lax.fori_loop(0, n_super, super_step, 0)
        plsc.subcore_barrier()
        pltpu.sync_copy(acc_sh.at[pl.ds(s * out_rows, out_rows), :],
                        out_hbm.at[c].at[pl.ds(s * out_rows, out_rows), :])

    return pl.kernel(
        body,
        out_type=jax.ShapeDtypeStruct((NC, rows, feat), jnp.float32),
        mesh=mesh,
        scratch_types=(
            [pltpu.VMEM((CK,), jnp.int32) for _ in range(2 * NB)]
            + [pltpu.VMEM((CK, feat), jnp.float32) for _ in range(NB)]
            + [pltpu.SemaphoreType.DMA for _ in range(4 * NB)]
            + [pltpu.VMEM_SHARED((rows, feat), jnp.float32)]
        ),
    )


def _first_layer_body(x_ref, w_ref, deg_ref, g_ref, dinv_ref):
    deg = deg_ref[0, :, 0:1] + deg_ref[1, :, 0:1] + 1.0
    dinv = lax.rsqrt(deg)
    g_ref[...] = jnp.dot(x_ref[...], w_ref[...],
                         preferred_element_type=jnp.float32) * dinv
    dinv_ref[...] = jnp.broadcast_to(dinv, dinv_ref.shape)


def _first_layer(x, w, deg_parts):
    n, d = x.shape
    return pl.pallas_call(
        _first_layer_body,
        grid=(n // RB,),
        in_specs=[
            pl.BlockSpec((RB, d), lambda i: (i, 0)),
            pl.BlockSpec((d, w.shape[1]), lambda i: (0, 0)),
            pl.BlockSpec((NC, RB, 128), lambda i: (0, i, 0)),
        ],
        out_specs=[
            pl.BlockSpec((RB, w.shape[1]), lambda i: (i, 0)),
            pl.BlockSpec((RB, 16), lambda i: (i, 0)),
        ],
        out_shape=[
            jax.ShapeDtypeStruct((n, w.shape[1]), jnp.float32),
            jax.ShapeDtypeStruct((n, 16), jnp.float32),
        ],
    )(x, w, deg_parts)


def _mid_layer_body(s_ref, gp_ref, dinv_ref, b_ref, w_ref, g_ref):
    dinv = dinv_ref[:, 0:1]
    act = jnp.maximum(
        dinv * (s_ref[0] + s_ref[1] + gp_ref[...]) + b_ref[...], 0.0)
    g_ref[...] = jnp.dot(act, w_ref[...],
                         preferred_element_type=jnp.float32) * dinv


def _mid_layer(s_parts, g_prev, dinv16, b, w):
    n, d = g_prev.shape
    return pl.pallas_call(
        _mid_layer_body,
        grid=(n // RB,),
        in_specs=[
            pl.BlockSpec((NC, RB, d), lambda i: (0, i, 0)),
            pl.BlockSpec((RB, d), lambda i: (i, 0)),
            pl.BlockSpec((RB, 16), lambda i: (i, 0)),
            pl.BlockSpec((1, d), lambda i: (0, 0)),
            pl.BlockSpec((d, w.shape[1]), lambda i: (0, 0)),
        ],
        out_specs=pl.BlockSpec((RB, w.shape[1]), lambda i: (i, 0)),
        out_shape=jax.ShapeDtypeStruct((n, w.shape[1]), jnp.float32),
    )(s_parts, g_prev, dinv16, b.reshape(1, -1), w)


def _pool_body(s_ref, gp_ref, dinv_ref, b_ref, batch_ref, fcw_ref, fcb_ref,
               out_ref, sums, cnts):
    i = pl.program_id(0)

    @pl.when(i == 0)
    def _():
        sums[...] = jnp.zeros_like(sums)
        cnts[...] = jnp.zeros_like(cnts)

    dinv = dinv_ref[:, 0:1]
    h = jnp.maximum(
        dinv * (s_ref[0] + s_ref[1] + gp_ref[...]) + b_ref[...], 0.0)
    onehot = (batch_ref[...] ==
              lax.broadcasted_iota(jnp.int32, (1, NG), 1)).astype(jnp.float32)
    dims = (((0,), (0,)), ((), ()))
    sums[...] += lax.dot_general(onehot, h, dims,
                                 preferred_element_type=jnp.float32)
    cnts[...] += lax.dot_general(onehot, jnp.ones_like(h), dims,
                                 preferred_element_type=jnp.float32)

    @pl.when(i == pl.num_programs(0) - 1)
    def _():
        pooled = sums[...] / jnp.maximum(cnts[...], 1.0)
        z = jnp.dot(pooled, fcw_ref[...],
                    preferred_element_type=jnp.float32) + fcb_ref[...]
        out_ref[...] = jax.nn.sigmoid(z)


def _pool(s_parts, g_prev, dinv16, b, batch, fcw, fcb):
    n, d = g_prev.shape
    return pl.pallas_call(
        _pool_body,
        grid=(n // RB,),
        in_specs=[
            pl.BlockSpec((NC, RB, d), lambda i: (0, i, 0)),
            pl.BlockSpec((RB, d), lambda i: (i, 0)),
            pl.BlockSpec((RB, 16), lambda i: (i, 0)),
            pl.BlockSpec((1, d), lambda i: (0, 0)),
            pl.BlockSpec((RB, 1), lambda i: (i, 0)),
            pl.BlockSpec((d, 1), lambda i: (0, 0)),
            pl.BlockSpec((1, 1), lambda i: (0, 0)),
        ],
        out_specs=pl.BlockSpec((NG, 1), lambda i: (0, 0)),
        out_shape=jax.ShapeDtypeStruct((NG, 1), jnp.float32),
        scratch_shapes=[
            pltpu.VMEM((NG, d), jnp.float32),
            pltpu.VMEM((NG, d), jnp.float32),
        ],
    )(s_parts, g_prev, dinv16, b.reshape(1, -1), batch.reshape(-1, 1),
      fcw, fcb.reshape(1, 1))


def kernel(x, edge_index, batch, W1, b1, W2, b2, W3, b3, fcW, fcb):
    n, d = x.shape
    e = edge_index.shape[1]
    e_pad = -(-e // (NW * CK * SB)) * (NW * CK * SB)
    pad = e_pad - e
    src = jnp.concatenate([edge_index[0], jnp.zeros((pad,), jnp.int32)])
    dst = jnp.concatenate([edge_index[1], jnp.full((pad,), n, jnp.int32)])

    deg_parts = _make_degree(n, e_pad)(dst.reshape(e_pad // CK, CK))
    m_total = e_pad // (NS * SB * CK)
    agg = _make_aggregate(n, d, e_pad, 9, m_total - 9)

    g1, dinv16 = _first_layer(x, W1, deg_parts)
    s1 = agg(g1, src, dst)
    g2 = _mid_layer(s1, g1, dinv16, b1, W2)
    s2 = agg(g2, src, dst)
    g3 = _mid_layer(s2, g2, dinv16, b2, W3)
    s3 = agg(g3, src, dst)
    return _pool(s3, g3, dinv16, b3, batch, fcW, fcb)
